# baseline (device time: 138958 ns/iter reference)
import os

import jax
import jax.numpy as jnp
from jax import lax
from jax.experimental import pallas as pl
from jax.experimental.pallas import tpu as pltpu

N = 4096
CH = 256
MAX_CHUNKS = 8


def _exchange_dest(d2):

    def body(d_ref, od_ref, sems):
        my_x = lax.axis_index("x")
        my_y = lax.axis_index("y")
        nbr = (my_x, 1 - my_y)

        barrier = pltpu.get_barrier_semaphore()
        pl.semaphore_signal(
            barrier, inc=1, device_id=nbr, device_id_type=pl.DeviceIdType.MESH
        )
        pl.semaphore_wait(barrier, 1)

        cd = pltpu.make_async_remote_copy(
            src_ref=d_ref,
            dst_ref=od_ref,
            send_sem=sems.at[0],
            recv_sem=sems.at[1],
            device_id=nbr,
            device_id_type=pl.DeviceIdType.MESH,
        )
        cd.start()
        cd.wait()

    return pl.pallas_call(
        body,
        out_shape=jax.ShapeDtypeStruct(d2.shape, d2.dtype),
        in_specs=[pl.BlockSpec(memory_space=pl.ANY)],
        out_specs=pl.BlockSpec(memory_space=pl.ANY),
        scratch_shapes=[pltpu.SemaphoreType.DMA((2,))],
        compiler_params=pltpu.CompilerParams(collective_id=1),
    )(d2)


def _a2av(x3, tpos, meta, seg_end):

    def body(x_ref, tp_ref, meta_ref, se_ref, buf_ref, ssy, rsy, ssx, rsx):
        my_x = lax.axis_index("x")
        my_y = lax.axis_index("y")
        ynbr = (my_x, 1 - my_y)
        dnbr = (1 - my_x, 1 - my_y)

        dst_off = meta_ref[0]
        recv_off = meta_ref[1]
        lo = meta_ref[2]
        len_y = meta_ref[3]
        n_sc = meta_ref[4]
        lo_n = meta_ref[5]
        len_n = meta_ref[6]
        n_rcy = meta_ref[7]
        lo_o = meta_ref[8]
        len_o = meta_ref[9]
        n_rcx = meta_ref[10]

        barrier = pltpu.get_barrier_semaphore()
        for nbr in (ynbr, dnbr):
            pl.semaphore_signal(
                barrier, inc=1, device_id=nbr,
                device_id_type=pl.DeviceIdType.MESH,
            )
        pl.semaphore_wait(barrier, 2)

        def scatter8(b, carry):
            blk = x_ref[pl.ds(8 * b, 8)]
            for t in range(8):
                buf_ref[pl.ds(tp_ref[8 * b + t], 1)] = blk[t : t + 1]
            return carry

        for c in range(MAX_CHUNKS):
            lax.fori_loop(se_ref[c], se_ref[c + 1], scatter8, 0)

            @pl.when(c < n_sc)
            def _():
                s = lo + jnp.minimum(c * CH, len_y - CH)
                pltpu.make_async_remote_copy(
                    src_ref=buf_ref.at[pl.ds(N + s, CH)],
                    dst_ref=buf_ref.at[pl.ds(dst_off + s, CH)],
                    send_sem=ssy.at[c],
                    recv_sem=rsy.at[c],
                    device_id=ynbr,
                    device_id_type=pl.DeviceIdType.MESH,
                ).start()
                pltpu.make_async_remote_copy(
                    src_ref=buf_ref.at[pl.ds(N + s, CH)],
                    dst_ref=buf_ref.at[pl.ds(dst_off + s, CH)],
                    send_sem=ssx.at[c],
                    recv_sem=rsx.at[c],
                    device_id=dnbr,
                    device_id_type=pl.DeviceIdType.MESH,
                ).start()

        lax.fori_loop(se_ref[MAX_CHUNKS], N // 8, scatter8, 0)

        for c in range(MAX_CHUNKS):
            @pl.when(c < n_sc)
            def _():
                pltpu.make_async_remote_copy(
                    src_ref=buf_ref.at[pl.ds(N, CH)],
                    dst_ref=buf_ref.at[pl.ds(0, CH)],
                    send_sem=ssy.at[c],
                    recv_sem=rsy.at[c],
                    device_id=ynbr,
                    device_id_type=pl.DeviceIdType.MESH,
                ).wait_send()
                pltpu.make_async_remote_copy(
                    src_ref=buf_ref.at[pl.ds(N, CH)],
                    dst_ref=buf_ref.at[pl.ds(0, CH)],
                    send_sem=ssx.at[c],
                    recv_sem=rsx.at[c],
                    device_id=dnbr,
                    device_id_type=pl.DeviceIdType.MESH,
                ).wait_send()

            @pl.when(c < n_rcy)
            def _():
                rs = recv_off + lo_n + jnp.minimum(c * CH, len_n - CH)
                pltpu.make_async_remote_copy(
                    src_ref=buf_ref.at[pl.ds(N, CH)],
                    dst_ref=buf_ref.at[pl.ds(rs, CH)],
                    send_sem=ssy.at[c],
                    recv_sem=rsy.at[c],
                    device_id=ynbr,
                    device_id_type=pl.DeviceIdType.MESH,
                ).wait_recv()

            @pl.when(c < n_rcx)
            def _():
                ro = recv_off + lo_o + jnp.minimum(c * CH, len_o - CH)
                pltpu.make_async_remote_copy(
                    src_ref=buf_ref.at[pl.ds(N, CH)],
                    dst_ref=buf_ref.at[pl.ds(ro, CH)],
                    send_sem=ssx.at[c],
                    recv_sem=rsx.at[c],
                    device_id=dnbr,
                    device_id_type=pl.DeviceIdType.MESH,
                ).wait_recv()

    return pl.pallas_call(
        body,
        out_shape=jax.ShapeDtypeStruct((2 * N, 8, 128), x3.dtype),
        in_specs=[
            pl.BlockSpec(memory_space=pltpu.VMEM),
            pl.BlockSpec(memory_space=pltpu.SMEM),
            pl.BlockSpec(memory_space=pltpu.SMEM),
            pl.BlockSpec(memory_space=pltpu.SMEM),
        ],
        out_specs=pl.BlockSpec(memory_space=pltpu.VMEM),
        scratch_shapes=[
            pltpu.SemaphoreType.DMA((MAX_CHUNKS,)),
            pltpu.SemaphoreType.DMA((MAX_CHUNKS,)),
            pltpu.SemaphoreType.DMA((MAX_CHUNKS,)),
            pltpu.SemaphoreType.DMA((MAX_CHUNKS,)),
        ],
        compiler_params=pltpu.CompilerParams(collective_id=0),
    )(x3, tpos, meta, seg_end)


def kernel(x, dest):
    r = lax.axis_index("y")
    mx = lax.axis_index("x")
    od = _exchange_dest(dest.reshape(8, N // 8)).reshape(N)

    m_keep = (dest == r).astype(jnp.int32)
    cs_keep = jnp.cumsum(m_keep)
    cs_send = jnp.cumsum(1 - m_keep)
    k_keep = cs_keep[-1]
    k_send = N - k_keep
    k_recv = jnp.sum((od == r).astype(jnp.int32))

    own_off = jnp.where(r == 0, 0, k_recv)
    recv_off = jnp.where(r == 0, k_keep, 0)
    dst_off = jnp.where(r == 0, 0, jnp.sum((od == 1 - r).astype(jnp.int32)))

    tpos = jnp.where(m_keep == 1, own_off + cs_keep - 1, N + cs_send - 1)

    h = (k_send + 1) // 2
    lo = jnp.where(mx == 0, 0, h)
    len_y = jnp.where(mx == 0, h, k_send - h)
    h_n = (k_recv + 1) // 2
    lo_n = jnp.where(mx == 0, 0, h_n)
    len_n = jnp.where(mx == 0, h_n, k_recv - h_n)
    lo_o = jnp.where(mx == 0, h_n, 0)
    len_o = k_recv - len_n

    n_sc = (len_y + CH - 1) // CH
    n_rcy = (len_n + CH - 1) // CH
    n_rcx = (len_o + CH - 1) // CH

    thr = lo + jnp.minimum(
        (jnp.arange(MAX_CHUNKS, dtype=jnp.int32) + 1) * CH, len_y
    )
    ends = jnp.sum(
        (cs_send[None, :] < thr[:, None]).astype(jnp.int32), axis=1
    ) + 1
    ends = (jnp.minimum(ends, N) + 7) // 8
    seg_end = jnp.concatenate([jnp.zeros((1,), jnp.int32), ends.astype(jnp.int32)])

    meta = jnp.stack(
        [dst_off, recv_off, lo, len_y, n_sc, lo_n, len_n, n_rcy, lo_o, len_o,
         n_rcx, 0]
    ).astype(jnp.int32)

    buf = _a2av(x.reshape(N, 8, 128), tpos.astype(jnp.int32), meta, seg_end)
    return buf[:N].reshape(N, 1024)


# device time: 103826 ns/iter; 1.3384x vs baseline; 1.3384x over previous
import os

import jax
import jax.numpy as jnp
from jax import lax
from jax.experimental import pallas as pl
from jax.experimental.pallas import tpu as pltpu

N = 4096
CH = 128
MAX_CHUNKS = 12


def _exchange_dest(d2):

    def body(d_ref, od_ref, sems):
        my_x = lax.axis_index("x")
        my_y = lax.axis_index("y")
        nbr = (my_x, 1 - my_y)

        barrier = pltpu.get_barrier_semaphore()
        pl.semaphore_signal(
            barrier, inc=1, device_id=nbr, device_id_type=pl.DeviceIdType.MESH
        )
        pl.semaphore_wait(barrier, 1)

        cd = pltpu.make_async_remote_copy(
            src_ref=d_ref,
            dst_ref=od_ref,
            send_sem=sems.at[0],
            recv_sem=sems.at[1],
            device_id=nbr,
            device_id_type=pl.DeviceIdType.MESH,
        )
        cd.start()
        cd.wait()

    return pl.pallas_call(
        body,
        out_shape=jax.ShapeDtypeStruct(d2.shape, d2.dtype),
        in_specs=[pl.BlockSpec(memory_space=pl.ANY)],
        out_specs=pl.BlockSpec(memory_space=pl.ANY),
        scratch_shapes=[pltpu.SemaphoreType.DMA((2,))],
        compiler_params=pltpu.CompilerParams(collective_id=1),
    )(d2)


def _a2av(x3, tpos, meta, seg_end):

    def body(x_ref, tp_ref, meta_ref, se_ref, buf_ref, ssy, rsy, ssx, rsx):
        my_x = lax.axis_index("x")
        my_y = lax.axis_index("y")
        ynbr = (my_x, 1 - my_y)
        xnbr = (1 - my_x, my_y)

        dst_off = meta_ref[0]
        recv_off = meta_ref[1]
        lo = meta_ref[2]
        len_y = meta_ref[3]
        n_sc = meta_ref[4]
        lo_n = meta_ref[5]
        len_n = meta_ref[6]
        n_rcy = meta_ref[7]
        lo_o = meta_ref[8]
        len_o = meta_ref[9]
        n_rcx = meta_ref[10]

        barrier = pltpu.get_barrier_semaphore()
        for nbr in (ynbr, xnbr):
            pl.semaphore_signal(
                barrier, inc=1, device_id=nbr,
                device_id_type=pl.DeviceIdType.MESH,
            )
        pl.semaphore_wait(barrier, 2)

        def scatter8(b, carry):
            blk = x_ref[pl.ds(8 * b, 8)]
            for t in range(8):
                buf_ref[pl.ds(tp_ref[8 * b + t], 1)] = blk[t : t + 1]
            return carry

        for c in range(MAX_CHUNKS):
            lax.fori_loop(se_ref[c], se_ref[c + 1], scatter8, 0)

            @pl.when(c < n_sc)
            def _():
                s = lo + jnp.minimum(c * CH, len_y - CH)
                pltpu.make_async_remote_copy(
                    src_ref=buf_ref.at[pl.ds(N + s, CH)],
                    dst_ref=buf_ref.at[pl.ds(dst_off + s, CH)],
                    send_sem=ssy.at[c],
                    recv_sem=rsy.at[c],
                    device_id=ynbr,
                    device_id_type=pl.DeviceIdType.MESH,
                ).start()

        lax.fori_loop(se_ref[MAX_CHUNKS], N // 8, scatter8, 0)

        for c in range(MAX_CHUNKS):
            @pl.when(c < n_rcy)
            def _():
                rs = recv_off + lo_n + jnp.minimum(c * CH, len_n - CH)
                pltpu.make_async_remote_copy(
                    src_ref=buf_ref.at[pl.ds(N, CH)],
                    dst_ref=buf_ref.at[pl.ds(rs, CH)],
                    send_sem=ssy.at[c],
                    recv_sem=rsy.at[c],
                    device_id=ynbr,
                    device_id_type=pl.DeviceIdType.MESH,
                ).wait_recv()
                pltpu.make_async_remote_copy(
                    src_ref=buf_ref.at[pl.ds(rs, CH)],
                    dst_ref=buf_ref.at[pl.ds(rs, CH)],
                    send_sem=ssx.at[c],
                    recv_sem=rsx.at[c],
                    device_id=xnbr,
                    device_id_type=pl.DeviceIdType.MESH,
                ).start()

        for c in range(MAX_CHUNKS):
            @pl.when(c < n_sc)
            def _():
                pltpu.make_async_remote_copy(
                    src_ref=buf_ref.at[pl.ds(N, CH)],
                    dst_ref=buf_ref.at[pl.ds(0, CH)],
                    send_sem=ssy.at[c],
                    recv_sem=rsy.at[c],
                    device_id=ynbr,
                    device_id_type=pl.DeviceIdType.MESH,
                ).wait_send()

            @pl.when(c < n_rcy)
            def _():
                pltpu.make_async_remote_copy(
                    src_ref=buf_ref.at[pl.ds(N, CH)],
                    dst_ref=buf_ref.at[pl.ds(0, CH)],
                    send_sem=ssx.at[c],
                    recv_sem=rsx.at[c],
                    device_id=xnbr,
                    device_id_type=pl.DeviceIdType.MESH,
                ).wait_send()

            @pl.when(c < n_rcx)
            def _():
                ro = recv_off + lo_o + jnp.minimum(c * CH, len_o - CH)
                pltpu.make_async_remote_copy(
                    src_ref=buf_ref.at[pl.ds(N, CH)],
                    dst_ref=buf_ref.at[pl.ds(ro, CH)],
                    send_sem=ssx.at[c],
                    recv_sem=rsx.at[c],
                    device_id=xnbr,
                    device_id_type=pl.DeviceIdType.MESH,
                ).wait_recv()

    return pl.pallas_call(
        body,
        out_shape=jax.ShapeDtypeStruct((2 * N, 8, 128), x3.dtype),
        in_specs=[
            pl.BlockSpec(memory_space=pltpu.VMEM),
            pl.BlockSpec(memory_space=pltpu.SMEM),
            pl.BlockSpec(memory_space=pltpu.SMEM),
            pl.BlockSpec(memory_space=pltpu.SMEM),
        ],
        out_specs=pl.BlockSpec(memory_space=pltpu.VMEM),
        scratch_shapes=[
            pltpu.SemaphoreType.DMA((MAX_CHUNKS,)),
            pltpu.SemaphoreType.DMA((MAX_CHUNKS,)),
            pltpu.SemaphoreType.DMA((MAX_CHUNKS,)),
            pltpu.SemaphoreType.DMA((MAX_CHUNKS,)),
        ],
        compiler_params=pltpu.CompilerParams(collective_id=0),
    )(x3, tpos, meta, seg_end)


def kernel(x, dest):
    r = lax.axis_index("y")
    mx = lax.axis_index("x")
    od = _exchange_dest(dest.reshape(8, N // 8)).reshape(N)

    m_keep = (dest == r).astype(jnp.int32)
    cs_keep = jnp.cumsum(m_keep)
    cs_send = jnp.cumsum(1 - m_keep)
    k_keep = cs_keep[-1]
    k_send = N - k_keep
    k_recv = jnp.sum((od == r).astype(jnp.int32))

    own_off = jnp.where(r == 0, 0, k_recv)
    recv_off = jnp.where(r == 0, k_keep, 0)
    dst_off = jnp.where(r == 0, 0, jnp.sum((od == 1 - r).astype(jnp.int32)))

    tpos = jnp.where(m_keep == 1, own_off + cs_keep - 1, N + cs_send - 1)

    h = (k_send + 1) // 2
    lo = jnp.where(mx == 0, 0, h)
    len_y = jnp.where(mx == 0, h, k_send - h)
    h_n = (k_recv + 1) // 2
    lo_n = jnp.where(mx == 0, 0, h_n)
    len_n = jnp.where(mx == 0, h_n, k_recv - h_n)
    lo_o = jnp.where(mx == 0, h_n, 0)
    len_o = k_recv - len_n

    n_sc = (len_y + CH - 1) // CH
    n_rcy = (len_n + CH - 1) // CH
    n_rcx = (len_o + CH - 1) // CH

    thr = lo + jnp.minimum(
        (jnp.arange(MAX_CHUNKS, dtype=jnp.int32) + 1) * CH, len_y
    )
    ends = jnp.sum(
        (cs_send[None, :] < thr[:, None]).astype(jnp.int32), axis=1
    ) + 1
    ends = (jnp.minimum(ends, N) + 7) // 8
    seg_end = jnp.concatenate([jnp.zeros((1,), jnp.int32), ends.astype(jnp.int32)])

    meta = jnp.stack(
        [dst_off, recv_off, lo, len_y, n_sc, lo_n, len_n, n_rcy, lo_o, len_o,
         n_rcx, 0]
    ).astype(jnp.int32)

    buf = _a2av(x.reshape(N, 8, 128), tpos.astype(jnp.int32), meta, seg_end)
    return buf[:N].reshape(N, 1024)


# device time: 101208 ns/iter; 1.3730x vs baseline; 1.0259x over previous
import os

import jax
import jax.numpy as jnp
from jax import lax
from jax.experimental import pallas as pl
from jax.experimental.pallas import tpu as pltpu

N = 4096
CH = 64
MAX_CHUNKS = 20


def _exchange_dest(d2):

    def body(d_ref, od_ref, sems):
        my_x = lax.axis_index("x")
        my_y = lax.axis_index("y")
        nbr = (my_x, 1 - my_y)

        barrier = pltpu.get_barrier_semaphore()
        pl.semaphore_signal(
            barrier, inc=1, device_id=nbr, device_id_type=pl.DeviceIdType.MESH
        )
        pl.semaphore_wait(barrier, 1)

        cd = pltpu.make_async_remote_copy(
            src_ref=d_ref,
            dst_ref=od_ref,
            send_sem=sems.at[0],
            recv_sem=sems.at[1],
            device_id=nbr,
            device_id_type=pl.DeviceIdType.MESH,
        )
        cd.start()
        cd.wait()

    return pl.pallas_call(
        body,
        out_shape=jax.ShapeDtypeStruct(d2.shape, d2.dtype),
        in_specs=[pl.BlockSpec(memory_space=pl.ANY)],
        out_specs=pl.BlockSpec(memory_space=pl.ANY),
        scratch_shapes=[pltpu.SemaphoreType.DMA((2,))],
        compiler_params=pltpu.CompilerParams(collective_id=1),
    )(d2)


def _a2av(x3, tpos, meta, seg_end):

    def body(x_ref, tp_ref, meta_ref, se_ref, buf_ref, ssy, rsy, ssx, rsx):
        my_x = lax.axis_index("x")
        my_y = lax.axis_index("y")
        ynbr = (my_x, 1 - my_y)
        xnbr = (1 - my_x, my_y)

        dst_off = meta_ref[0]
        recv_off = meta_ref[1]
        lo = meta_ref[2]
        len_y = meta_ref[3]
        n_sc = meta_ref[4]
        lo_n = meta_ref[5]
        len_n = meta_ref[6]
        n_rcy = meta_ref[7]
        lo_o = meta_ref[8]
        len_o = meta_ref[9]
        n_rcx = meta_ref[10]

        barrier = pltpu.get_barrier_semaphore()
        for nbr in (ynbr, xnbr):
            pl.semaphore_signal(
                barrier, inc=1, device_id=nbr,
                device_id_type=pl.DeviceIdType.MESH,
            )
        pl.semaphore_wait(barrier, 2)

        def scatter8(b, carry):
            blk = x_ref[pl.ds(8 * b, 8)]
            for t in range(8):
                buf_ref[pl.ds(tp_ref[8 * b + t], 1)] = blk[t : t + 1]
            return carry

        for c in range(MAX_CHUNKS):
            lax.fori_loop(se_ref[c], se_ref[c + 1], scatter8, 0)

            @pl.when(c < n_sc)
            def _():
                s = lo + jnp.minimum(c * CH, len_y - CH)
                pltpu.make_async_remote_copy(
                    src_ref=buf_ref.at[pl.ds(N + s, CH)],
                    dst_ref=buf_ref.at[pl.ds(dst_off + s, CH)],
                    send_sem=ssy.at[c],
                    recv_sem=rsy.at[c],
                    device_id=ynbr,
                    device_id_type=pl.DeviceIdType.MESH,
                ).start()

        lax.fori_loop(se_ref[MAX_CHUNKS], N // 8, scatter8, 0)

        for c in range(MAX_CHUNKS):
            @pl.when(c < n_rcy)
            def _():
                rs = recv_off + lo_n + jnp.minimum(c * CH, len_n - CH)
                pltpu.make_async_remote_copy(
                    src_ref=buf_ref.at[pl.ds(N, CH)],
                    dst_ref=buf_ref.at[pl.ds(rs, CH)],
                    send_sem=ssy.at[c],
                    recv_sem=rsy.at[c],
                    device_id=ynbr,
                    device_id_type=pl.DeviceIdType.MESH,
                ).wait_recv()
                pltpu.make_async_remote_copy(
                    src_ref=buf_ref.at[pl.ds(rs, CH)],
                    dst_ref=buf_ref.at[pl.ds(rs, CH)],
                    send_sem=ssx.at[c],
                    recv_sem=rsx.at[c],
                    device_id=xnbr,
                    device_id_type=pl.DeviceIdType.MESH,
                ).start()

        for c in range(MAX_CHUNKS):
            @pl.when(c < n_sc)
            def _():
                pltpu.make_async_remote_copy(
                    src_ref=buf_ref.at[pl.ds(N, CH)],
                    dst_ref=buf_ref.at[pl.ds(0, CH)],
                    send_sem=ssy.at[c],
                    recv_sem=rsy.at[c],
                    device_id=ynbr,
                    device_id_type=pl.DeviceIdType.MESH,
                ).wait_send()

            @pl.when(c < n_rcy)
            def _():
                pltpu.make_async_remote_copy(
                    src_ref=buf_ref.at[pl.ds(N, CH)],
                    dst_ref=buf_ref.at[pl.ds(0, CH)],
                    send_sem=ssx.at[c],
                    recv_sem=rsx.at[c],
                    device_id=xnbr,
                    device_id_type=pl.DeviceIdType.MESH,
                ).wait_send()

            @pl.when(c < n_rcx)
            def _():
                ro = recv_off + lo_o + jnp.minimum(c * CH, len_o - CH)
                pltpu.make_async_remote_copy(
                    src_ref=buf_ref.at[pl.ds(N, CH)],
                    dst_ref=buf_ref.at[pl.ds(ro, CH)],
                    send_sem=ssx.at[c],
                    recv_sem=rsx.at[c],
                    device_id=xnbr,
                    device_id_type=pl.DeviceIdType.MESH,
                ).wait_recv()

    return pl.pallas_call(
        body,
        out_shape=jax.ShapeDtypeStruct((2 * N, 8, 128), x3.dtype),
        in_specs=[
            pl.BlockSpec(memory_space=pltpu.VMEM),
            pl.BlockSpec(memory_space=pltpu.SMEM),
            pl.BlockSpec(memory_space=pltpu.SMEM),
            pl.BlockSpec(memory_space=pltpu.SMEM),
        ],
        out_specs=pl.BlockSpec(memory_space=pltpu.VMEM),
        scratch_shapes=[
            pltpu.SemaphoreType.DMA((MAX_CHUNKS,)),
            pltpu.SemaphoreType.DMA((MAX_CHUNKS,)),
            pltpu.SemaphoreType.DMA((MAX_CHUNKS,)),
            pltpu.SemaphoreType.DMA((MAX_CHUNKS,)),
        ],
        compiler_params=pltpu.CompilerParams(collective_id=0),
    )(x3, tpos, meta, seg_end)


def kernel(x, dest):
    r = lax.axis_index("y")
    mx = lax.axis_index("x")
    od = _exchange_dest(dest.reshape(8, N // 8)).reshape(N)

    m_keep = (dest == r).astype(jnp.int32)
    cs_keep = jnp.cumsum(m_keep)
    cs_send = jnp.cumsum(1 - m_keep)
    k_keep = cs_keep[-1]
    k_send = N - k_keep
    k_recv = jnp.sum((od == r).astype(jnp.int32))

    own_off = jnp.where(r == 0, 0, k_recv)
    recv_off = jnp.where(r == 0, k_keep, 0)
    dst_off = jnp.where(r == 0, 0, jnp.sum((od == 1 - r).astype(jnp.int32)))

    tpos = jnp.where(m_keep == 1, own_off + cs_keep - 1, N + cs_send - 1)

    h = (k_send + 1) // 2
    lo = jnp.where(mx == 0, 0, h)
    len_y = jnp.where(mx == 0, h, k_send - h)
    h_n = (k_recv + 1) // 2
    lo_n = jnp.where(mx == 0, 0, h_n)
    len_n = jnp.where(mx == 0, h_n, k_recv - h_n)
    lo_o = jnp.where(mx == 0, h_n, 0)
    len_o = k_recv - len_n

    n_sc = (len_y + CH - 1) // CH
    n_rcy = (len_n + CH - 1) // CH
    n_rcx = (len_o + CH - 1) // CH

    thr = lo + jnp.minimum(
        (jnp.arange(MAX_CHUNKS, dtype=jnp.int32) + 1) * CH, len_y
    )
    ends = jnp.sum(
        (cs_send[None, :] < thr[:, None]).astype(jnp.int32), axis=1
    ) + 1
    ends = (jnp.minimum(ends, N) + 7) // 8
    seg_end = jnp.concatenate([jnp.zeros((1,), jnp.int32), ends.astype(jnp.int32)])

    meta = jnp.stack(
        [dst_off, recv_off, lo, len_y, n_sc, lo_n, len_n, n_rcy, lo_o, len_o,
         n_rcx, 0]
    ).astype(jnp.int32)

    buf = _a2av(x.reshape(N, 8, 128), tpos.astype(jnp.int32), meta, seg_end)
    return buf[:N].reshape(N, 1024)


# device time: 99945 ns/iter; 1.3903x vs baseline; 1.0126x over previous
import os

import jax
import jax.numpy as jnp
from jax import lax
from jax.experimental import pallas as pl
from jax.experimental.pallas import tpu as pltpu

N = 4096
CH = 64
MAX_CHUNKS = 20
SROWS = 3072 + 8


def _exchange_dest(d2):

    def body(d_ref, od_ref, sems):
        my_x = lax.axis_index("x")
        my_y = lax.axis_index("y")
        nbr = (my_x, 1 - my_y)

        barrier = pltpu.get_barrier_semaphore()
        pl.semaphore_signal(
            barrier, inc=1, device_id=nbr, device_id_type=pl.DeviceIdType.MESH
        )
        pl.semaphore_wait(barrier, 1)

        cd = pltpu.make_async_remote_copy(
            src_ref=d_ref,
            dst_ref=od_ref,
            send_sem=sems.at[0],
            recv_sem=sems.at[1],
            device_id=nbr,
            device_id_type=pl.DeviceIdType.MESH,
        )
        cd.start()
        cd.wait()

    return pl.pallas_call(
        body,
        out_shape=jax.ShapeDtypeStruct(d2.shape, d2.dtype),
        in_specs=[pl.BlockSpec(memory_space=pl.ANY)],
        out_specs=pl.BlockSpec(memory_space=pl.ANY),
        scratch_shapes=[pltpu.SemaphoreType.DMA((2,))],
        compiler_params=pltpu.CompilerParams(collective_id=1),
    )(d2)


def _a2av(x3, tpos, meta, seg_end):

    def body(x_ref, tp_ref, meta_ref, se_ref, buf_ref, ssy, rsy, ssx, rsx):
        my_x = lax.axis_index("x")
        my_y = lax.axis_index("y")
        ynbr = (my_x, 1 - my_y)
        xnbr = (1 - my_x, my_y)

        dst_off = meta_ref[0]
        recv_off = meta_ref[1]
        lo = meta_ref[2]
        len_y = meta_ref[3]
        n_sc = meta_ref[4]
        lo_n = meta_ref[5]
        len_n = meta_ref[6]
        n_rcy = meta_ref[7]
        lo_o = meta_ref[8]
        len_o = meta_ref[9]
        n_rcx = meta_ref[10]

        barrier = pltpu.get_barrier_semaphore()
        for nbr in (ynbr, xnbr):
            pl.semaphore_signal(
                barrier, inc=1, device_id=nbr,
                device_id_type=pl.DeviceIdType.MESH,
            )
        pl.semaphore_wait(barrier, 2)

        def scatter8(b, carry):
            blk = x_ref[pl.ds(8 * b, 8)]
            for t in range(8):
                buf_ref[pl.ds(tp_ref[8 * b + t], 1)] = blk[t : t + 1]
            return carry

        for c in range(MAX_CHUNKS):
            lax.fori_loop(se_ref[c], se_ref[c + 1], scatter8, 0)

            @pl.when(c < n_sc)
            def _():
                s = lo + jnp.minimum(c * CH, len_y - CH)
                pltpu.make_async_remote_copy(
                    src_ref=buf_ref.at[pl.ds(N + s, CH)],
                    dst_ref=buf_ref.at[pl.ds(dst_off + s, CH)],
                    send_sem=ssy.at[c],
                    recv_sem=rsy.at[c],
                    device_id=ynbr,
                    device_id_type=pl.DeviceIdType.MESH,
                ).start()

        lax.fori_loop(se_ref[MAX_CHUNKS], N // 8, scatter8, 0)

        for c in range(MAX_CHUNKS):
            @pl.when(c < n_rcy)
            def _():
                rs = recv_off + lo_n + jnp.minimum(c * CH, len_n - CH)
                pltpu.make_async_remote_copy(
                    src_ref=buf_ref.at[pl.ds(N, CH)],
                    dst_ref=buf_ref.at[pl.ds(rs, CH)],
                    send_sem=ssy.at[c],
                    recv_sem=rsy.at[c],
                    device_id=ynbr,
                    device_id_type=pl.DeviceIdType.MESH,
                ).wait_recv()
                pltpu.make_async_remote_copy(
                    src_ref=buf_ref.at[pl.ds(rs, CH)],
                    dst_ref=buf_ref.at[pl.ds(rs, CH)],
                    send_sem=ssx.at[c],
                    recv_sem=rsx.at[c],
                    device_id=xnbr,
                    device_id_type=pl.DeviceIdType.MESH,
                ).start()

        for c in range(MAX_CHUNKS):
            @pl.when(c < n_sc)
            def _():
                pltpu.make_async_remote_copy(
                    src_ref=buf_ref.at[pl.ds(N, CH)],
                    dst_ref=buf_ref.at[pl.ds(0, CH)],
                    send_sem=ssy.at[c],
                    recv_sem=rsy.at[c],
                    device_id=ynbr,
                    device_id_type=pl.DeviceIdType.MESH,
                ).wait_send()

            @pl.when(c < n_rcy)
            def _():
                pltpu.make_async_remote_copy(
                    src_ref=buf_ref.at[pl.ds(N, CH)],
                    dst_ref=buf_ref.at[pl.ds(0, CH)],
                    send_sem=ssx.at[c],
                    recv_sem=rsx.at[c],
                    device_id=xnbr,
                    device_id_type=pl.DeviceIdType.MESH,
                ).wait_send()

            @pl.when(c < n_rcx)
            def _():
                ro = recv_off + lo_o + jnp.minimum(c * CH, len_o - CH)
                pltpu.make_async_remote_copy(
                    src_ref=buf_ref.at[pl.ds(N, CH)],
                    dst_ref=buf_ref.at[pl.ds(ro, CH)],
                    send_sem=ssx.at[c],
                    recv_sem=rsx.at[c],
                    device_id=xnbr,
                    device_id_type=pl.DeviceIdType.MESH,
                ).wait_recv()

    return pl.pallas_call(
        body,
        out_shape=jax.ShapeDtypeStruct((N + SROWS, 8, 128), x3.dtype),
        in_specs=[
            pl.BlockSpec(memory_space=pltpu.VMEM),
            pl.BlockSpec(memory_space=pltpu.SMEM),
            pl.BlockSpec(memory_space=pltpu.SMEM),
            pl.BlockSpec(memory_space=pltpu.SMEM),
        ],
        out_specs=pl.BlockSpec(memory_space=pltpu.VMEM),
        scratch_shapes=[
            pltpu.SemaphoreType.DMA((MAX_CHUNKS,)),
            pltpu.SemaphoreType.DMA((MAX_CHUNKS,)),
            pltpu.SemaphoreType.DMA((MAX_CHUNKS,)),
            pltpu.SemaphoreType.DMA((MAX_CHUNKS,)),
        ],
        compiler_params=pltpu.CompilerParams(collective_id=0),
    )(x3, tpos, meta, seg_end)


def kernel(x, dest):
    r = lax.axis_index("y")
    mx = lax.axis_index("x")
    od = _exchange_dest(dest.reshape(8, N // 8)).reshape(N)

    m_keep = (dest == r).astype(jnp.int32)
    cs_keep = jnp.cumsum(m_keep)
    cs_send = jnp.cumsum(1 - m_keep)
    k_keep = cs_keep[-1]
    k_send = N - k_keep
    k_recv = jnp.sum((od == r).astype(jnp.int32))

    own_off = jnp.where(r == 0, 0, k_recv)
    recv_off = jnp.where(r == 0, k_keep, 0)
    dst_off = jnp.where(r == 0, 0, jnp.sum((od == 1 - r).astype(jnp.int32)))

    tpos = jnp.where(m_keep == 1, own_off + cs_keep - 1, N + cs_send - 1)

    h = (k_send + 1) // 2
    lo = jnp.where(mx == 0, 0, h)
    len_y = jnp.where(mx == 0, h, k_send - h)
    h_n = (k_recv + 1) // 2
    lo_n = jnp.where(mx == 0, 0, h_n)
    len_n = jnp.where(mx == 0, h_n, k_recv - h_n)
    lo_o = jnp.where(mx == 0, h_n, 0)
    len_o = k_recv - len_n

    n_sc = (len_y + CH - 1) // CH
    n_rcy = (len_n + CH - 1) // CH
    n_rcx = (len_o + CH - 1) // CH

    thr = lo + jnp.minimum(
        (jnp.arange(MAX_CHUNKS, dtype=jnp.int32) + 1) * CH, len_y
    )
    ends = jnp.sum(
        (cs_send[None, :] < thr[:, None]).astype(jnp.int32), axis=1
    ) + 1
    ends = (jnp.minimum(ends, N) + 7) // 8
    seg_end = jnp.concatenate([jnp.zeros((1,), jnp.int32), ends.astype(jnp.int32)])

    meta = jnp.stack(
        [dst_off, recv_off, lo, len_y, n_sc, lo_n, len_n, n_rcy, lo_o, len_o,
         n_rcx, 0]
    ).astype(jnp.int32)

    buf = _a2av(x.reshape(N, 8, 128), tpos.astype(jnp.int32), meta, seg_end)
    return buf[:N].reshape(N, 1024)


# device time: 99833 ns/iter; 1.3919x vs baseline; 1.0011x over previous
import jax
import jax.numpy as jnp
from jax import lax
from jax.experimental import pallas as pl
from jax.experimental.pallas import tpu as pltpu

N = 4096
CH = 64
MAX_CHUNKS = 20
SROWS = 3072 + 8


def _exchange_dest(d2):

    def body(d_ref, od_ref, sems):
        my_x = lax.axis_index("x")
        my_y = lax.axis_index("y")
        nbr = (my_x, 1 - my_y)

        barrier = pltpu.get_barrier_semaphore()
        pl.semaphore_signal(
            barrier, inc=1, device_id=nbr, device_id_type=pl.DeviceIdType.MESH
        )
        pl.semaphore_wait(barrier, 1)

        cd = pltpu.make_async_remote_copy(
            src_ref=d_ref,
            dst_ref=od_ref,
            send_sem=sems.at[0],
            recv_sem=sems.at[1],
            device_id=nbr,
            device_id_type=pl.DeviceIdType.MESH,
        )
        cd.start()
        cd.wait()

    return pl.pallas_call(
        body,
        out_shape=jax.ShapeDtypeStruct(d2.shape, d2.dtype),
        in_specs=[pl.BlockSpec(memory_space=pl.ANY)],
        out_specs=pl.BlockSpec(memory_space=pl.ANY),
        scratch_shapes=[pltpu.SemaphoreType.DMA((2,))],
        compiler_params=pltpu.CompilerParams(collective_id=1),
    )(d2)


def _a2av(x3, tpos, meta, seg_end):

    def body(x_ref, tp_ref, meta_ref, se_ref, buf_ref, ssy, rsy, ssx, rsx):
        my_x = lax.axis_index("x")
        my_y = lax.axis_index("y")
        ynbr = (my_x, 1 - my_y)
        xnbr = (1 - my_x, my_y)

        dst_off = meta_ref[0]
        recv_off = meta_ref[1]
        lo = meta_ref[2]
        len_y = meta_ref[3]
        n_sc = meta_ref[4]
        lo_n = meta_ref[5]
        len_n = meta_ref[6]
        n_rcy = meta_ref[7]
        lo_o = meta_ref[8]
        len_o = meta_ref[9]
        n_rcx = meta_ref[10]

        barrier = pltpu.get_barrier_semaphore()
        for nbr in (ynbr, xnbr):
            pl.semaphore_signal(
                barrier, inc=1, device_id=nbr,
                device_id_type=pl.DeviceIdType.MESH,
            )
        pl.semaphore_wait(barrier, 2)

        def scatter8(b, carry):
            blk = x_ref[pl.ds(8 * b, 8)]
            for t in range(8):
                buf_ref[pl.ds(tp_ref[8 * b + t], 1)] = blk[t : t + 1]
            return carry

        for c in range(MAX_CHUNKS):
            lax.fori_loop(se_ref[c], se_ref[c + 1], scatter8, 0)

            @pl.when(c < n_sc)
            def _():
                s = lo + jnp.minimum(c * CH, len_y - CH)
                pltpu.make_async_remote_copy(
                    src_ref=buf_ref.at[pl.ds(N + s, CH)],
                    dst_ref=buf_ref.at[pl.ds(dst_off + s, CH)],
                    send_sem=ssy.at[c],
                    recv_sem=rsy.at[c],
                    device_id=ynbr,
                    device_id_type=pl.DeviceIdType.MESH,
                ).start()

        lax.fori_loop(se_ref[MAX_CHUNKS], N // 8, scatter8, 0)

        for c in range(MAX_CHUNKS):
            @pl.when(c < n_rcy)
            def _():
                rs = recv_off + lo_n + jnp.minimum(c * CH, len_n - CH)
                pltpu.make_async_remote_copy(
                    src_ref=buf_ref.at[pl.ds(N, CH)],
                    dst_ref=buf_ref.at[pl.ds(rs, CH)],
                    send_sem=ssy.at[c],
                    recv_sem=rsy.at[c],
                    device_id=ynbr,
                    device_id_type=pl.DeviceIdType.MESH,
                ).wait_recv()
                pltpu.make_async_remote_copy(
                    src_ref=buf_ref.at[pl.ds(rs, CH)],
                    dst_ref=buf_ref.at[pl.ds(rs, CH)],
                    send_sem=ssx.at[c],
                    recv_sem=rsx.at[c],
                    device_id=xnbr,
                    device_id_type=pl.DeviceIdType.MESH,
                ).start()

        for c in range(MAX_CHUNKS):
            @pl.when(c < n_sc)
            def _():
                pltpu.make_async_remote_copy(
                    src_ref=buf_ref.at[pl.ds(N, CH)],
                    dst_ref=buf_ref.at[pl.ds(0, CH)],
                    send_sem=ssy.at[c],
                    recv_sem=rsy.at[c],
                    device_id=ynbr,
                    device_id_type=pl.DeviceIdType.MESH,
                ).wait_send()

            @pl.when(c < n_rcy)
            def _():
                pltpu.make_async_remote_copy(
                    src_ref=buf_ref.at[pl.ds(N, CH)],
                    dst_ref=buf_ref.at[pl.ds(0, CH)],
                    send_sem=ssx.at[c],
                    recv_sem=rsx.at[c],
                    device_id=xnbr,
                    device_id_type=pl.DeviceIdType.MESH,
                ).wait_send()

            @pl.when(c < n_rcx)
            def _():
                ro = recv_off + lo_o + jnp.minimum(c * CH, len_o - CH)
                pltpu.make_async_remote_copy(
                    src_ref=buf_ref.at[pl.ds(N, CH)],
                    dst_ref=buf_ref.at[pl.ds(ro, CH)],
                    send_sem=ssx.at[c],
                    recv_sem=rsx.at[c],
                    device_id=xnbr,
                    device_id_type=pl.DeviceIdType.MESH,
                ).wait_recv()

    return pl.pallas_call(
        body,
        out_shape=jax.ShapeDtypeStruct((N + SROWS, 8, 128), x3.dtype),
        in_specs=[
            pl.BlockSpec(memory_space=pltpu.VMEM),
            pl.BlockSpec(memory_space=pltpu.SMEM),
            pl.BlockSpec(memory_space=pltpu.SMEM),
            pl.BlockSpec(memory_space=pltpu.SMEM),
        ],
        out_specs=pl.BlockSpec(memory_space=pltpu.VMEM),
        scratch_shapes=[
            pltpu.SemaphoreType.DMA((MAX_CHUNKS,)),
            pltpu.SemaphoreType.DMA((MAX_CHUNKS,)),
            pltpu.SemaphoreType.DMA((MAX_CHUNKS,)),
            pltpu.SemaphoreType.DMA((MAX_CHUNKS,)),
        ],
        compiler_params=pltpu.CompilerParams(collective_id=0),
    )(x3, tpos, meta, seg_end)


def kernel(x, dest):
    r = lax.axis_index("y")
    mx = lax.axis_index("x")
    od = _exchange_dest(dest.reshape(8, N // 8)).reshape(N)

    m_keep = (dest == r).astype(jnp.int32)
    cs_keep = jnp.cumsum(m_keep)
    cs_send = jnp.cumsum(1 - m_keep)
    k_keep = cs_keep[-1]
    k_send = N - k_keep
    k_recv = jnp.sum((od == r).astype(jnp.int32))

    own_off = jnp.where(r == 0, 0, k_recv)
    recv_off = jnp.where(r == 0, k_keep, 0)
    dst_off = jnp.where(r == 0, 0, jnp.sum((od == 1 - r).astype(jnp.int32)))

    tpos = jnp.where(m_keep == 1, own_off + cs_keep - 1, N + cs_send - 1)

    h = (k_send + 1) // 2
    lo = jnp.where(mx == 0, 0, h)
    len_y = jnp.where(mx == 0, h, k_send - h)
    h_n = (k_recv + 1) // 2
    lo_n = jnp.where(mx == 0, 0, h_n)
    len_n = jnp.where(mx == 0, h_n, k_recv - h_n)
    lo_o = jnp.where(mx == 0, h_n, 0)
    len_o = k_recv - len_n

    n_sc = (len_y + CH - 1) // CH
    n_rcy = (len_n + CH - 1) // CH
    n_rcx = (len_o + CH - 1) // CH

    thr = lo + jnp.minimum(
        (jnp.arange(MAX_CHUNKS, dtype=jnp.int32) + 1) * CH, len_y
    )
    ends = jnp.sum(
        (cs_send[None, :] < thr[:, None]).astype(jnp.int32), axis=1
    ) + 1
    ends = (jnp.minimum(ends, N) + 7) // 8
    seg_end = jnp.concatenate([jnp.zeros((1,), jnp.int32), ends.astype(jnp.int32)])

    meta = jnp.stack(
        [dst_off, recv_off, lo, len_y, n_sc, lo_n, len_n, n_rcy, lo_o, len_o,
         n_rcx, 0]
    ).astype(jnp.int32)

    buf = _a2av(x.reshape(N, 8, 128), tpos.astype(jnp.int32), meta, seg_end)
    return buf[:N].reshape(N, 1024)
